# baseline (speedup 1.0000x reference)
import jax
import jax.numpy as jnp
from jax.experimental import pallas as pl
from jax.experimental.pallas import tpu as pltpu

N_ROWS = 100000
D_NODE = 140
BLK = 10000
NBLK = N_ROWS // BLK
T_STEPS = 200
T_CHUNK = T_STEPS // NBLK


def _fused_kernel(x_hbm, w_ref, bg_ref, cfg_ref, wih0_ref, whh0_ref, b0_ref,
                  wih1_ref, whh1_ref, b1_ref, wfc_ref, bfc_ref,
                  out_ref, buf_ref, g0_ref, sem):

    def copy_in(i, slot):
        return pltpu.make_async_copy(
            x_hbm.at[pl.ds(i * BLK, BLK), :], buf_ref.at[slot], sem.at[slot])

    copy_in(0, 0).start()

    g0_ref[...] = (jnp.dot(cfg_ref[...], wih0_ref[...],
                           preferred_element_type=jnp.float32)
                   + b0_ref[...])

    w = w_ref[...]
    bg = bg_ref[0, 0]
    whh0 = whh0_ref[...]
    wih1 = wih1_ref[...]
    whh1 = whh1_ref[...]
    b1 = b1_ref[...]
    half = jnp.float32(0.5)

    def lstm_chunk(i, h0, c0, h1, c1, skip_first_l1):
        slab = g0_ref[pl.ds(i * T_CHUNK, T_CHUNK), :]
        for k in range(T_CHUNK):
            th0 = jnp.tanh(slab[k:k + 1, :] + h0 * whh0)
            if not (skip_first_l1 and k == 0):
                th1 = jnp.tanh(h0 * wih1 + h1 * whh1 + b1)
                i1 = half + half * th1[0, 0]
                f1 = half + half * th1[0, 1]
                o1 = half + half * th1[0, 3]
                c1 = f1 * c1 + i1 * th1[0, 2]
                h1 = o1 * jnp.tanh(c1)
            i0 = half + half * th0[0, 0]
            f0 = half + half * th0[0, 1]
            o0 = half + half * th0[0, 3]
            c0 = f0 * c0 + i0 * th0[0, 2]
            h0 = o0 * jnp.tanh(c0)
        return h0, c0, h1, c1

    def block_tail(slot, x_idx, acc):
        copy_in(x_idx, slot).wait()
        x = buf_ref[slot]
        y = jnp.dot(x, w, preferred_element_type=jnp.float32)
        return acc + (jnp.sum(jnp.maximum(y[:, 0] + bg, 0.0))
                      + jnp.sum(y[:, 1]))

    def block(i, carry):
        acc, h0, c0, h1, c1 = carry
        slot = jax.lax.rem(i, 2)

        @pl.when(i + 1 < NBLK)
        def _prefetch():
            copy_in(i + 1, 1 - slot).start()

        h0, c0, h1, c1 = lstm_chunk(i, h0, c0, h1, c1, False)
        return (block_tail(slot, i, acc), h0, c0, h1, c1)

    z = jnp.float32(0.0)
    copy_in(1, 1).start()
    h0, c0, h1, c1 = lstm_chunk(0, z, z, z, z, True)
    acc = block_tail(0, 0, z)
    acc, h0, c0, h1, c1 = jax.lax.fori_loop(1, NBLK, block,
                                            (acc, h0, c0, h1, c1))

    th1 = jnp.tanh(h0 * wih1 + h1 * whh1 + b1)
    i1 = half + half * th1[0, 0]
    f1 = half + half * th1[0, 1]
    o1 = half + half * th1[0, 3]
    c1 = f1 * c1 + i1 * th1[0, 2]
    h1 = o1 * jnp.tanh(c1)

    gnn = acc * jnp.float32(1.0 / N_ROWS)
    wfc = wfc_ref[...]
    res = gnn * wfc[0, 0] + h1 * wfc[0, 1] + bfc_ref[0, 0]
    out_ref[...] = jnp.reshape(res, (1, 1))


@jax.jit
def _run(node_feat, cfg, w_col, bg, wih0_t, whh0r, b0r, wih1r, whh1r, b1r,
         wfc, bfc):
    vm = lambda: pl.BlockSpec(memory_space=pltpu.MemorySpace.VMEM)
    return pl.pallas_call(
        _fused_kernel,
        in_specs=[
            pl.BlockSpec(memory_space=pltpu.MemorySpace.HBM),
            vm(), vm(), vm(), vm(), vm(), vm(), vm(), vm(), vm(), vm(), vm(),
        ],
        out_specs=pl.BlockSpec(memory_space=pltpu.MemorySpace.VMEM),
        out_shape=jax.ShapeDtypeStruct((1, 1), jnp.float32),
        scratch_shapes=[
            pltpu.VMEM((2, BLK, D_NODE), jnp.float32),
            pltpu.VMEM((T_STEPS, 4), jnp.float32),
            pltpu.SemaphoreType.DMA((2,)),
        ],
    )(node_feat, w_col, bg, cfg, wih0_t, whh0r, b0r, wih1r, whh1r, b1r,
      wfc, bfc)


def kernel(node_feat, edge_index, config_feat, W_gnn, b_gnn, Wih0, Whh0,
           bih0, bhh0, Wih1, Whh1, bih1, bhh1, W_fc, b_fc):
    cfg = config_feat.reshape(config_feat.shape[1], config_feat.shape[2])
    sel = jnp.zeros((D_NODE, 1), jnp.float32).at[0, 0].set(1.0)
    w_col = jnp.concatenate([W_gnn.reshape(D_NODE, 1), sel], axis=1)
    bg = b_gnn.reshape(1, 1)
    gsc = jnp.array([[0.5, 0.5, 1.0, 0.5]], jnp.float32)
    wih0_t = Wih0.T * gsc
    whh0r = Whh0.T.reshape(1, 4) * gsc
    b0r = (bih0 + bhh0).reshape(1, 4) * gsc
    wih1r = Wih1.T.reshape(1, 4) * gsc
    whh1r = Whh1.T.reshape(1, 4) * gsc
    b1r = (bih1 + bhh1).reshape(1, 4) * gsc
    wfc = W_fc.reshape(1, 2)
    bfc = b_fc.reshape(1, 1)
    return _run(node_feat, cfg, w_col, bg, wih0_t, whh0r, b0r, wih1r, whh1r,
                b1r, wfc, bfc)


# speedup vs baseline: 1.0849x; 1.0849x over previous
import jax
import jax.numpy as jnp
from jax.experimental import pallas as pl
from jax.experimental.pallas import tpu as pltpu

N_ROWS = 100000
D_NODE = 140
BLK = 10000
NBLK = N_ROWS // BLK
T_STEPS = 200
T_CHUNK = T_STEPS // NBLK


def _fused_kernel(x_hbm, w_ref, bg_ref, cfg_ref, wih0_ref, whh0_ref, b0_ref,
                  wih1_ref, whh1_ref, b1_ref, wfc_ref, bfc_ref,
                  out_ref, buf_ref, g0_ref, sem):

    def copy_in(i, slot):
        return pltpu.make_async_copy(
            x_hbm.at[pl.ds(i * BLK, BLK), :], buf_ref.at[slot], sem.at[slot])

    copy_in(0, 0).start()

    g0_ref[...] = (jnp.dot(cfg_ref[...], wih0_ref[...],
                           preferred_element_type=jnp.float32)
                   + b0_ref[...])

    w = w_ref[...]
    bg = bg_ref[0, 0]
    whh0 = whh0_ref[...]
    wih1 = wih1_ref[...]
    whh1 = whh1_ref[...]
    b1 = b1_ref[...]
    half = jnp.float32(0.5)

    def lstm_chunk(i, h0, c0, h1, c1):
        slab = g0_ref[pl.ds(i * T_CHUNK, T_CHUNK), :]
        for k in range(T_CHUNK):
            t_glob = i * T_CHUNK + k
            th0 = jnp.tanh(slab[k:k + 1, :] + h0 * whh0)
            th1 = jnp.tanh(h0 * wih1 + h1 * whh1 + b1)
            i0 = half + half * th0[0, 0]
            f0 = half + half * th0[0, 1]
            o0 = half + half * th0[0, 3]
            i1 = half + half * th1[0, 0]
            f1 = half + half * th1[0, 1]
            o1 = half + half * th1[0, 3]
            c0 = f0 * c0 + i0 * th0[0, 2]
            c1n = f1 * c1 + i1 * th1[0, 2]
            h0n = o0 * jnp.tanh(c0)
            h1n = o1 * jnp.tanh(c1n)
            keep = (t_glob >= 1).astype(jnp.float32)
            c1 = keep * c1n + (1.0 - keep) * c1
            h1 = keep * h1n + (1.0 - keep) * h1
            h0 = h0n
        return h0, c0, h1, c1

    def block(i, carry):
        acc, h0, c0, h1, c1 = carry
        slot = jax.lax.rem(i, 2)

        @pl.when(i + 1 < NBLK)
        def _prefetch():
            copy_in(i + 1, 1 - slot).start()

        h0, c0, h1, c1 = lstm_chunk(i, h0, c0, h1, c1)

        copy_in(i, slot).wait()
        x = buf_ref[slot]
        y = jnp.dot(x, w, preferred_element_type=jnp.float32)
        part = (jnp.sum(jnp.maximum(y[:, 0] + bg, 0.0)) + jnp.sum(y[:, 1]))
        return (acc + part, h0, c0, h1, c1)

    z = jnp.float32(0.0)
    acc, h0, c0, h1, c1 = jax.lax.fori_loop(0, NBLK, block, (z, z, z, z, z))

    th1 = jnp.tanh(h0 * wih1 + h1 * whh1 + b1)
    i1 = half + half * th1[0, 0]
    f1 = half + half * th1[0, 1]
    o1 = half + half * th1[0, 3]
    c1 = f1 * c1 + i1 * th1[0, 2]
    h1 = o1 * jnp.tanh(c1)

    gnn = acc * jnp.float32(1.0 / N_ROWS)
    wfc = wfc_ref[...]
    res = gnn * wfc[0, 0] + h1 * wfc[0, 1] + bfc_ref[0, 0]
    out_ref[...] = jnp.reshape(res, (1, 1))


@jax.jit
def _run(node_feat, cfg, w_col, bg, wih0_t, whh0r, b0r, wih1r, whh1r, b1r,
         wfc, bfc):
    vm = lambda: pl.BlockSpec(memory_space=pltpu.MemorySpace.VMEM)
    return pl.pallas_call(
        _fused_kernel,
        in_specs=[
            pl.BlockSpec(memory_space=pltpu.MemorySpace.HBM),
            vm(), vm(), vm(), vm(), vm(), vm(), vm(), vm(), vm(), vm(), vm(),
        ],
        out_specs=pl.BlockSpec(memory_space=pltpu.MemorySpace.VMEM),
        out_shape=jax.ShapeDtypeStruct((1, 1), jnp.float32),
        scratch_shapes=[
            pltpu.VMEM((2, BLK, D_NODE), jnp.float32),
            pltpu.VMEM((T_STEPS, 4), jnp.float32),
            pltpu.SemaphoreType.DMA((2,)),
        ],
    )(node_feat, w_col, bg, cfg, wih0_t, whh0r, b0r, wih1r, whh1r, b1r,
      wfc, bfc)


def kernel(node_feat, edge_index, config_feat, W_gnn, b_gnn, Wih0, Whh0,
           bih0, bhh0, Wih1, Whh1, bih1, bhh1, W_fc, b_fc):
    cfg = config_feat.reshape(config_feat.shape[1], config_feat.shape[2])
    sel = jnp.zeros((D_NODE, 1), jnp.float32).at[0, 0].set(1.0)
    w_col = jnp.concatenate([W_gnn.reshape(D_NODE, 1), sel], axis=1)
    bg = b_gnn.reshape(1, 1)
    gsc = jnp.array([[0.5, 0.5, 1.0, 0.5]], jnp.float32)
    wih0_t = Wih0.T * gsc
    whh0r = Whh0.T.reshape(1, 4) * gsc
    b0r = (bih0 + bhh0).reshape(1, 4) * gsc
    wih1r = Wih1.T.reshape(1, 4) * gsc
    whh1r = Whh1.T.reshape(1, 4) * gsc
    b1r = (bih1 + bhh1).reshape(1, 4) * gsc
    wfc = W_fc.reshape(1, 2)
    bfc = b_fc.reshape(1, 1)
    return _run(node_feat, cfg, w_col, bg, wih0_t, whh0r, b0r, wih1r, whh1r,
                b1r, wfc, bfc)
